# baseline (speedup 1.0000x reference)
import jax
import jax.numpy as jnp
from jax.experimental import pallas as pl

_NQ = 8192
_TQ = 128
_NT = _NQ // _TQ
_CK = 128
_NC = _NQ // _CK
_NB = 4
_ND = 2 * _NB


def _nn_body(q_ref, k_ref, min_ref, idx_ref, sum_ref):
    t = pl.program_id(1)
    q = q_ref[0]
    qx = q[:, 0:1]
    qy = q[:, 1:2]
    qz = q[:, 2:3]

    def step(j, carry):
        mmin, tid = carry
        k = k_ref[0, :, pl.ds(j * _CK, _CK)]
        dx = qx - k[0:1, :]
        dy = qy - k[1:2, :]
        dz = qz - k[2:3, :]
        d = dx * dx + dy * dy + dz * dz
        upd = d < mmin
        mmin = jnp.minimum(mmin, d)
        tid = jnp.where(upd, jnp.float32(j), tid)
        return mmin, tid

    mmin = jnp.full((_TQ, _CK), jnp.inf, jnp.float32)
    tid = jnp.zeros((_TQ, _CK), jnp.float32)
    mmin, tid = jax.lax.fori_loop(0, _NC, step, (mmin, tid), unroll=8)

    m = jnp.min(mmin, axis=1)
    lane = jax.lax.broadcasted_iota(jnp.int32, (_TQ, _CK), 1).astype(jnp.float32)
    cand = tid * jnp.float32(_CK) + lane
    idxf = jnp.min(jnp.where(mmin == m[:, None], cand, jnp.float32(2 * _NQ)),
                   axis=1)
    min_ref[0, 0, :] = m
    idx_ref[0, 0, :] = idxf.astype(jnp.int32)

    @pl.when(t == 0)
    def _():
        sum_ref[0, 0, :] = jnp.zeros((_TQ,), jnp.float32)

    sum_ref[0, 0, :] += m


def kernel(pred_points, true_points):
    qs = jnp.concatenate([pred_points, true_points], axis=0)
    ks = jnp.concatenate([true_points, pred_points], axis=0)
    ks = ks.transpose(0, 2, 1)

    grid = (_ND, _NT)
    mins, idxs, sums = pl.pallas_call(
        _nn_body,
        grid=grid,
        in_specs=[
            pl.BlockSpec((1, _TQ, 3), lambda b, t: (b, t, 0)),
            pl.BlockSpec((1, 3, _NQ), lambda b, t: (b, 0, 0)),
        ],
        out_specs=[
            pl.BlockSpec((1, 1, _TQ), lambda b, t: (b * _NT + t, 0, 0)),
            pl.BlockSpec((1, 1, _TQ), lambda b, t: (b * _NT + t, 0, 0)),
            pl.BlockSpec((1, 1, _TQ), lambda b, t: (b, 0, 0)),
        ],
        out_shape=[
            jax.ShapeDtypeStruct((_ND * _NT, 1, _TQ), jnp.float32),
            jax.ShapeDtypeStruct((_ND * _NT, 1, _TQ), jnp.int32),
            jax.ShapeDtypeStruct((_ND, 1, _TQ), jnp.float32),
        ],
    )(qs, ks)

    loss = jnp.sum(sums) / (_NB * _NQ)
    idxs = idxs.reshape(_ND, _NQ)
    return loss, idxs[:_NB], idxs[_NB:]
